# baseline (speedup 1.0000x reference)
import functools

import jax
import jax.numpy as jnp
from jax import lax
from jax.experimental import pallas as pl
from jax.experimental.pallas import tpu as pltpu
from jax.experimental.pallas import tpu_sc as plsc

NC = 2
NS = 16
NW = NC * NS

B = 8
T = 4
BT = B * T
N = 1601
D = 1280
CK = 16
NCH = 100
NPW = NCH * BT // NW
VPT = D // 16
UNR = 8


def _sc_fused(x3, g3, lp, p_arr, m_arr, c_arr):
  mesh = plsc.VectorSubcoreMesh(core_axis_name="c", subcore_axis_name="s")

  @functools.partial(
      pl.kernel,
      mesh=mesh,
      out_type=jax.ShapeDtypeStruct((BT, N, D), jnp.float32),
      scratch_types=[
          pltpu.VMEM((CK, D), jnp.float32),
          pltpu.VMEM((CK, D), jnp.float32),
          pltpu.VMEM((CK, D), jnp.float32),
          pltpu.VMEM((CK, D), jnp.float32),
          pltpu.VMEM((CK, D), jnp.float32),
          pltpu.VMEM((BT, 128), jnp.int32),
          pltpu.VMEM((BT, 128), jnp.float32),
          pltpu.VMEM((1, 128), jnp.float32),
          pltpu.SemaphoreType.DMA,
          pltpu.SemaphoreType.DMA,
          pltpu.SemaphoreType.DMA,
          pltpu.SemaphoreType.DMA,
          pltpu.SemaphoreType.DMA,
          pltpu.SemaphoreType.DMA,
      ],
  )
  def k(x_hbm, g_hbm, l_hbm, p_hbm, m_hbm, c_hbm, out_hbm,
        xb0, xb1, gb0, gb1, lb, pb, mb, cb,
        sx0, sx1, sg0, sg1, so0, so1):
    wid = lax.axis_index("s") * NC + lax.axis_index("c")
    pltpu.sync_copy(p_hbm, pb)
    pltpu.sync_copy(m_hbm, mb)
    pltpu.sync_copy(c_hbm, cb)
    cvec = cb[0, pl.ds(0, 16)]
    q0 = wid * NPW

    def drain(buf, sem):
      pltpu.make_async_copy(x_hbm.at[0, pl.ds(0, CK), :], buf, sem).wait()

    def drain_out(buf, sem):
      pltpu.make_async_copy(buf, out_hbm.at[0, pl.ds(0, CK), :], sem).wait()

    def issue(q, xbuf, gbuf, sx, sg):
      bt = lax.rem(q, BT)
      n0 = (q // BT) * CK
      p = pb[bt, pl.ds(0, 16)][0]
      pltpu.async_copy(g_hbm.at[p, pl.ds(n0, CK), :], gbuf, sg)
      pltpu.async_copy(x_hbm.at[bt, pl.ds(n0, CK), :], xbuf, sx)

    def store(q, xbuf, so):
      bt = lax.rem(q, BT)
      n0 = (q // BT) * CK
      pltpu.async_copy(xbuf, out_hbm.at[bt, pl.ds(n0, CK), :], so)

    def load_local(lc):
      n0 = lc * CK
      pltpu.sync_copy(l_hbm.at[pl.ds(n0, CK), :], lb)

      def i_body(i, cc):
        def j_body(j, cc2):
          s = pl.ds(j * 16, 16)
          lb[i, s] = cvec * lb[i, s]
          return cc2
        return lax.fori_loop(0, VPT, j_body, cc)
      lax.fori_loop(0, CK, i_body, 0)

    def fused_rows(xbuf, gbuf, mv):
      def i_body(i, cc):
        def jj_body(jj, cc2):
          for u in range(UNR):
            s = pl.ds(jj * (16 * UNR) + u * 16, 16)
            xbuf[i, s] = xbuf[i, s] + lb[i, s] + mv * gbuf[i, s]
          return cc2
        return lax.fori_loop(0, VPT // UNR, jj_body, cc)
      lax.fori_loop(0, CK, i_body, 0)

    issue(q0, xb0, gb0, sx0, sg0)

    def pair_body(it2, carry2):
      qa = q0 + 2 * it2
      qb = qa + 1

      @pl.when(jnp.logical_or(it2 == 0, lax.rem(qa, BT) == 0))
      def _():
        load_local(qa // BT)

      @pl.when(it2 >= 1)
      def _():
        drain_out(xb1, so1)
      issue(qb, xb1, gb1, sx1, sg1)

      drain(xb0, sx0)
      drain(gb0, sg0)
      fused_rows(xb0, gb0, mb[lax.rem(qa, BT), pl.ds(0, 16)])
      store(qa, xb0, so0)

      @pl.when(it2 <= (NPW // 2 - 2))
      def _():
        drain_out(xb0, so0)
        issue(qa + 2, xb0, gb0, sx0, sg0)

      drain(xb1, sx1)
      drain(gb1, sg1)
      fused_rows(xb1, gb1, mb[lax.rem(qb, BT), pl.ds(0, 16)])
      store(qb, xb1, so1)
      return carry2

    lax.fori_loop(0, NPW // 2, pair_body, 0)
    drain_out(xb0, so0)
    drain_out(xb1, so1)

    @pl.when(wid == 0)
    def _():
      n0 = NCH * CK
      pltpu.sync_copy(l_hbm.at[pl.ds(n0, 1), :], lb.at[pl.ds(0, 1), :])

      def bt_tail(bt, carry2):
        p = pb[bt, pl.ds(0, 16)][0]
        cp = pltpu.async_copy(g_hbm.at[p, pl.ds(n0, 1), :],
                              gb0.at[pl.ds(0, 1), :], sg0)
        pltpu.sync_copy(x_hbm.at[bt, pl.ds(n0, 1), :], xb0.at[pl.ds(0, 1), :])
        cp.wait()
        mv = mb[bt, pl.ds(0, 16)]

        def j_body(j, c4):
          s = pl.ds(j * 16, 16)
          xb0[0, s] = xb0[0, s] + cvec * lb[0, s] + mv * gb0[0, s]
          return c4

        lax.fori_loop(0, VPT, j_body, 0)
        pltpu.sync_copy(xb0.at[pl.ds(0, 1), :],
                        out_hbm.at[bt, pl.ds(n0, 1), :])
        return carry2

      lax.fori_loop(0, BT, bt_tail, 0)

  return k(x3, g3, lp, p_arr, m_arr, c_arr)


def kernel(x, aspect_ratio, local_pe, global_pe, gate):
  b, t, n, d = x.shape
  g2 = jnp.tanh(gate[0].astype(jnp.float32))
  ar = aspect_ratio.astype(jnp.int32)
  h = ar[:, 0]
  w = ar[:, 1]
  wsafe = jnp.maximum(w, 1)
  tt = jnp.arange(T, dtype=jnp.int32)
  rows = tt[None, :] // wsafe[:, None]
  cols = tt[None, :] % wsafe[:, None]
  plane = (rows * T + cols).reshape(BT)
  valid = (tt[None, :] < (h * w)[:, None]).reshape(BT)
  p_arr = jnp.tile(plane.reshape(BT, 1), (1, 128))
  m_arr = jnp.tile((g2 * valid.astype(jnp.float32)).reshape(BT, 1), (1, 128))
  c_arr = jnp.tile((1.0 - g2).reshape(1, 1), (1, 128))
  x3 = x.reshape(BT, N, D)
  g3 = global_pe.reshape(T * T, N, D)
  out = _sc_fused(x3, g3, local_pe, p_arr, m_arr, c_arr)
  return out.reshape(b, t, n, d)


# speedup vs baseline: 1.6028x; 1.6028x over previous
import functools

import jax
import jax.numpy as jnp
from jax import lax
from jax.experimental import pallas as pl
from jax.experimental.pallas import tpu as pltpu
from jax.experimental.pallas import tpu_sc as plsc

NC = 2
NS = 16
NW = NC * NS

B = 8
T = 4
BT = B * T
N = 1601
D = 1280
S = 1216
TB = 64
CK = 16
NCH = (1600 - S) // CK
NPW = NCH * BT // NW
VPT = D // 16
UNR = 8


def _tc_fused(x, lp, g3, plane, m_arr, c_arr):
  def body(plane_ref, c_ref, m_ref, x_ref, l_ref, g_ref, o_ref):
    b = pl.program_id(1)
    t = pl.program_id(2)
    m = m_ref[b * T + t]
    add = c_ref[0] * l_ref[...] + m * g_ref[0]
    o_ref[...] = x_ref[...] + add[None, None]

  grid_spec = pltpu.PrefetchScalarGridSpec(
      num_scalar_prefetch=1,
      grid=(S // TB, B, T),
      in_specs=[
          pl.BlockSpec(memory_space=pltpu.SMEM),
          pl.BlockSpec(memory_space=pltpu.SMEM),
          pl.BlockSpec((1, 1, TB, D), lambda i, j, k, pref: (j, k, i, 0)),
          pl.BlockSpec((TB, D), lambda i, j, k, pref: (i, 0)),
          pl.BlockSpec((1, TB, D),
                       lambda i, j, k, pref: (pref[j * T + k], i, 0)),
      ],
      out_specs=pl.BlockSpec((1, 1, TB, D), lambda i, j, k, pref: (j, k, i, 0)),
  )
  return pl.pallas_call(
      body,
      grid_spec=grid_spec,
      out_shape=jax.ShapeDtypeStruct((B, T, S, D), jnp.float32),
  )(plane, c_arr, m_arr, x, lp, g3)


def _sc_fused(x3, g3, lp, p_arr, m_arr, c_arr):
  mesh = plsc.VectorSubcoreMesh(core_axis_name="c", subcore_axis_name="s")

  @functools.partial(
      pl.kernel,
      mesh=mesh,
      out_type=jax.ShapeDtypeStruct((BT, N - S, D), jnp.float32),
      scratch_types=[
          pltpu.VMEM((CK, D), jnp.float32),
          pltpu.VMEM((CK, D), jnp.float32),
          pltpu.VMEM((CK, D), jnp.float32),
          pltpu.VMEM((CK, D), jnp.float32),
          pltpu.VMEM((CK, D), jnp.float32),
          pltpu.VMEM((BT, 128), jnp.int32),
          pltpu.VMEM((BT, 128), jnp.float32),
          pltpu.VMEM((1, 128), jnp.float32),
          pltpu.SemaphoreType.DMA,
          pltpu.SemaphoreType.DMA,
          pltpu.SemaphoreType.DMA,
          pltpu.SemaphoreType.DMA,
          pltpu.SemaphoreType.DMA,
          pltpu.SemaphoreType.DMA,
      ],
  )
  def k(x_hbm, g_hbm, l_hbm, p_hbm, m_hbm, c_hbm, out_hbm,
        xb0, xb1, gb0, gb1, lb, pb, mb, cb,
        sx0, sx1, sg0, sg1, so0, so1):
    wid = lax.axis_index("s") * NC + lax.axis_index("c")
    pltpu.sync_copy(p_hbm, pb)
    pltpu.sync_copy(m_hbm, mb)
    pltpu.sync_copy(c_hbm, cb)
    cvec = cb[0, pl.ds(0, 16)]
    q0 = wid * NPW

    def drain(buf, sem):
      pltpu.make_async_copy(x_hbm.at[0, pl.ds(0, CK), :], buf, sem).wait()

    def drain_out(buf, sem):
      pltpu.make_async_copy(buf, out_hbm.at[0, pl.ds(0, CK), :], sem).wait()

    def issue(q, xbuf, gbuf, sx, sg):
      bt = lax.rem(q, BT)
      ns = (q // BT) * CK
      p = pb[bt, pl.ds(0, 16)][0]
      pltpu.async_copy(g_hbm.at[p, pl.ds(S + ns, CK), :], gbuf, sg)
      pltpu.async_copy(x_hbm.at[bt, pl.ds(S + ns, CK), :], xbuf, sx)

    def store(q, xbuf, so):
      bt = lax.rem(q, BT)
      ns = (q // BT) * CK
      pltpu.async_copy(xbuf, out_hbm.at[bt, pl.ds(ns, CK), :], so)

    def load_local(lc):
      pltpu.sync_copy(l_hbm.at[pl.ds(S + lc * CK, CK), :], lb)

      def i_body(i, cc):
        def j_body(j, cc2):
          s = pl.ds(j * 16, 16)
          lb[i, s] = cvec * lb[i, s]
          return cc2
        return lax.fori_loop(0, VPT, j_body, cc)
      lax.fori_loop(0, CK, i_body, 0)

    def fused_rows(xbuf, gbuf, mv):
      def i_body(i, cc):
        def jj_body(jj, cc2):
          for u in range(UNR):
            s = pl.ds(jj * (16 * UNR) + u * 16, 16)
            xbuf[i, s] = xbuf[i, s] + lb[i, s] + mv * gbuf[i, s]
          return cc2
        return lax.fori_loop(0, VPT // UNR, jj_body, cc)
      lax.fori_loop(0, CK, i_body, 0)

    issue(q0, xb0, gb0, sx0, sg0)

    def pair_body(it2, carry2):
      qa = q0 + 2 * it2
      qb = qa + 1

      @pl.when(jnp.logical_or(it2 == 0, lax.rem(qa, BT) == 0))
      def _():
        load_local(qa // BT)

      @pl.when(it2 >= 1)
      def _():
        drain_out(xb1, so1)
      issue(qb, xb1, gb1, sx1, sg1)

      drain(xb0, sx0)
      drain(gb0, sg0)
      fused_rows(xb0, gb0, mb[lax.rem(qa, BT), pl.ds(0, 16)])
      store(qa, xb0, so0)

      @pl.when(it2 <= (NPW // 2 - 2))
      def _():
        drain_out(xb0, so0)
        issue(qa + 2, xb0, gb0, sx0, sg0)

      drain(xb1, sx1)
      drain(gb1, sg1)
      fused_rows(xb1, gb1, mb[lax.rem(qb, BT), pl.ds(0, 16)])
      store(qb, xb1, so1)
      return carry2

    lax.fori_loop(0, NPW // 2, pair_body, 0)
    drain_out(xb0, so0)
    drain_out(xb1, so1)

    @pl.when(wid == 0)
    def _():
      n0 = 1600
      pltpu.sync_copy(l_hbm.at[pl.ds(n0, 1), :], lb.at[pl.ds(0, 1), :])

      def bt_tail(bt, carry2):
        p = pb[bt, pl.ds(0, 16)][0]
        cp = pltpu.async_copy(g_hbm.at[p, pl.ds(n0, 1), :],
                              gb0.at[pl.ds(0, 1), :], sg0)
        pltpu.sync_copy(x_hbm.at[bt, pl.ds(n0, 1), :], xb0.at[pl.ds(0, 1), :])
        cp.wait()
        mv = mb[bt, pl.ds(0, 16)]

        def j_body(j, c4):
          s = pl.ds(j * 16, 16)
          xb0[0, s] = xb0[0, s] + cvec * lb[0, s] + mv * gb0[0, s]
          return c4

        lax.fori_loop(0, VPT, j_body, 0)
        pltpu.sync_copy(xb0.at[pl.ds(0, 1), :],
                        out_hbm.at[bt, pl.ds(n0 - S, 1), :])
        return carry2

      lax.fori_loop(0, BT, bt_tail, 0)

  return k(x3, g3, lp, p_arr, m_arr, c_arr)


def kernel(x, aspect_ratio, local_pe, global_pe, gate):
  b, t, n, d = x.shape
  g2 = jnp.tanh(gate[0].astype(jnp.float32))
  ar = aspect_ratio.astype(jnp.int32)
  h = ar[:, 0]
  w = ar[:, 1]
  wsafe = jnp.maximum(w, 1)
  tt = jnp.arange(T, dtype=jnp.int32)
  rows = tt[None, :] // wsafe[:, None]
  cols = tt[None, :] % wsafe[:, None]
  plane = (rows * T + cols).reshape(BT)
  valid = (tt[None, :] < (h * w)[:, None]).reshape(BT)
  mflat = g2 * valid.astype(jnp.float32)
  p_arr = jnp.tile(plane.reshape(BT, 1), (1, 128))
  m_arr = jnp.tile(mflat.reshape(BT, 1), (1, 128))
  c_arr = jnp.tile((1.0 - g2).reshape(1, 1), (1, 128))
  x3 = x.reshape(BT, N, D)
  g3 = global_pe.reshape(T * T, N, D)
  out_tc = _tc_fused(x, local_pe, g3, plane, m_arr[:, 0],
                     c_arr[0, :1])
  out_sc = _sc_fused(x3, g3, local_pe, p_arr, m_arr, c_arr)
  out = jnp.concatenate(
      [out_tc, out_sc.reshape(b, t, N - S, d)], axis=2)
  return out


# speedup vs baseline: 2.0288x; 1.2658x over previous
import functools

import jax
import jax.numpy as jnp
from jax import lax
from jax.experimental import pallas as pl
from jax.experimental.pallas import tpu as pltpu
from jax.experimental.pallas import tpu_sc as plsc

NC = 2
NS = 16
NW = NC * NS

B = 8
T = 4
BT = B * T
N = 1601
D = 1280
S = 1216
TB = 64
CK = 16
NCH = (1600 - S) // CK
NPW = NCH * BT // NW
VPT = D // 16
UNR = 8


def _tc_fused(x, lp, g3, plane, m_arr, c_arr):
  def body(plane_ref, c_ref, m_ref, x_ref, l_ref, g_ref, o_ref):
    b = pl.program_id(1)
    t = pl.program_id(2)
    m = m_ref[b * T + t]
    add = c_ref[0] * l_ref[...] + m * g_ref[0]
    o_ref[...] = x_ref[...] + add[None, None]

  grid_spec = pltpu.PrefetchScalarGridSpec(
      num_scalar_prefetch=1,
      grid=(S // TB, B, T),
      in_specs=[
          pl.BlockSpec(memory_space=pltpu.SMEM),
          pl.BlockSpec(memory_space=pltpu.SMEM),
          pl.BlockSpec((1, 1, TB, D), lambda i, j, k, pref: (j, k, i, 0)),
          pl.BlockSpec((TB, D), lambda i, j, k, pref: (i, 0)),
          pl.BlockSpec((1, TB, D),
                       lambda i, j, k, pref: (pref[j * T + k], i, 0)),
      ],
      out_specs=pl.BlockSpec((1, 1, TB, D), lambda i, j, k, pref: (j, k, i, 0)),
  )
  return pl.pallas_call(
      body,
      grid_spec=grid_spec,
      out_shape=jax.ShapeDtypeStruct((B, T, S, D), jnp.float32),
  )(plane, c_arr, m_arr, x, lp, g3)


def _tc_assemble(tc_out, sc_out):
  NB = S // TB

  def body(a_ref, b_ref, o_ref):
    i = pl.program_id(0)

    @pl.when(i < NB)
    def _():
      o_ref[...] = a_ref[...]

    @pl.when(i >= NB)
    def _():
      o_ref[...] = b_ref[...]

  grid = (pl.cdiv(N, TB), B, T)
  return pl.pallas_call(
      body,
      grid=grid,
      in_specs=[
          pl.BlockSpec((1, 1, TB, D),
                       lambda i, j, k: (j, k, jnp.minimum(i, NB - 1), 0)),
          pl.BlockSpec((1, 1, TB, D),
                       lambda i, j, k: (j, k, jnp.maximum(i - NB, 0), 0)),
      ],
      out_specs=pl.BlockSpec((1, 1, TB, D), lambda i, j, k: (j, k, i, 0)),
      out_shape=jax.ShapeDtypeStruct((B, T, N, D), jnp.float32),
  )(tc_out, sc_out)


def _sc_fused(x3, g3, lp, p_arr, m_arr, c_arr):
  mesh = plsc.VectorSubcoreMesh(core_axis_name="c", subcore_axis_name="s")

  @functools.partial(
      pl.kernel,
      mesh=mesh,
      out_type=jax.ShapeDtypeStruct((BT, N - S, D), jnp.float32),
      scratch_types=[
          pltpu.VMEM((CK, D), jnp.float32),
          pltpu.VMEM((CK, D), jnp.float32),
          pltpu.VMEM((CK, D), jnp.float32),
          pltpu.VMEM((CK, D), jnp.float32),
          pltpu.VMEM((CK, D), jnp.float32),
          pltpu.VMEM((BT, 128), jnp.int32),
          pltpu.VMEM((BT, 128), jnp.float32),
          pltpu.VMEM((1, 128), jnp.float32),
          pltpu.SemaphoreType.DMA,
          pltpu.SemaphoreType.DMA,
          pltpu.SemaphoreType.DMA,
          pltpu.SemaphoreType.DMA,
          pltpu.SemaphoreType.DMA,
          pltpu.SemaphoreType.DMA,
      ],
  )
  def k(x_hbm, g_hbm, l_hbm, p_hbm, m_hbm, c_hbm, out_hbm,
        xb0, xb1, gb0, gb1, lb, pb, mb, cb,
        sx0, sx1, sg0, sg1, so0, so1):
    wid = lax.axis_index("s") * NC + lax.axis_index("c")
    pltpu.sync_copy(p_hbm, pb)
    pltpu.sync_copy(m_hbm, mb)
    pltpu.sync_copy(c_hbm, cb)
    cvec = cb[0, pl.ds(0, 16)]
    q0 = wid * NPW

    def drain(buf, sem):
      pltpu.make_async_copy(x_hbm.at[0, pl.ds(0, CK), :], buf, sem).wait()

    def drain_out(buf, sem):
      pltpu.make_async_copy(buf, out_hbm.at[0, pl.ds(0, CK), :], sem).wait()

    def issue(q, xbuf, gbuf, sx, sg):
      bt = lax.rem(q, BT)
      ns = (q // BT) * CK
      p = pb[bt, pl.ds(0, 16)][0]
      pltpu.async_copy(g_hbm.at[p, pl.ds(S + ns, CK), :], gbuf, sg)
      pltpu.async_copy(x_hbm.at[bt, pl.ds(S + ns, CK), :], xbuf, sx)

    def store(q, xbuf, so):
      bt = lax.rem(q, BT)
      ns = (q // BT) * CK
      pltpu.async_copy(xbuf, out_hbm.at[bt, pl.ds(ns, CK), :], so)

    def load_local(lc):
      pltpu.sync_copy(l_hbm.at[pl.ds(S + lc * CK, CK), :], lb)

      def i_body(i, cc):
        def j_body(j, cc2):
          s = pl.ds(j * 16, 16)
          lb[i, s] = cvec * lb[i, s]
          return cc2
        return lax.fori_loop(0, VPT, j_body, cc)
      lax.fori_loop(0, CK, i_body, 0)

    def fused_rows(xbuf, gbuf, mv):
      def i_body(i, cc):
        def jj_body(jj, cc2):
          for u in range(UNR):
            s = pl.ds(jj * (16 * UNR) + u * 16, 16)
            xbuf[i, s] = xbuf[i, s] + lb[i, s] + mv * gbuf[i, s]
          return cc2
        return lax.fori_loop(0, VPT // UNR, jj_body, cc)
      lax.fori_loop(0, CK, i_body, 0)

    issue(q0, xb0, gb0, sx0, sg0)

    def pair_body(it2, carry2):
      qa = q0 + 2 * it2
      qb = qa + 1

      @pl.when(jnp.logical_or(it2 == 0, lax.rem(qa, BT) == 0))
      def _():
        load_local(qa // BT)

      @pl.when(it2 >= 1)
      def _():
        drain_out(xb1, so1)
      issue(qb, xb1, gb1, sx1, sg1)

      drain(xb0, sx0)
      drain(gb0, sg0)
      fused_rows(xb0, gb0, mb[lax.rem(qa, BT), pl.ds(0, 16)])
      store(qa, xb0, so0)

      @pl.when(it2 <= (NPW // 2 - 2))
      def _():
        drain_out(xb0, so0)
        issue(qa + 2, xb0, gb0, sx0, sg0)

      drain(xb1, sx1)
      drain(gb1, sg1)
      fused_rows(xb1, gb1, mb[lax.rem(qb, BT), pl.ds(0, 16)])
      store(qb, xb1, so1)
      return carry2

    lax.fori_loop(0, NPW // 2, pair_body, 0)
    drain_out(xb0, so0)
    drain_out(xb1, so1)

    @pl.when(wid == 0)
    def _():
      n0 = 1600
      pltpu.sync_copy(l_hbm.at[pl.ds(n0, 1), :], lb.at[pl.ds(0, 1), :])

      def bt_tail(bt, carry2):
        p = pb[bt, pl.ds(0, 16)][0]
        cp = pltpu.async_copy(g_hbm.at[p, pl.ds(n0, 1), :],
                              gb0.at[pl.ds(0, 1), :], sg0)
        pltpu.sync_copy(x_hbm.at[bt, pl.ds(n0, 1), :], xb0.at[pl.ds(0, 1), :])
        cp.wait()
        mv = mb[bt, pl.ds(0, 16)]

        def j_body(j, c4):
          s = pl.ds(j * 16, 16)
          xb0[0, s] = xb0[0, s] + cvec * lb[0, s] + mv * gb0[0, s]
          return c4

        lax.fori_loop(0, VPT, j_body, 0)
        pltpu.sync_copy(xb0.at[pl.ds(0, 1), :],
                        out_hbm.at[bt, pl.ds(n0 - S, 1), :])
        return carry2

      lax.fori_loop(0, BT, bt_tail, 0)

  return k(x3, g3, lp, p_arr, m_arr, c_arr)


def kernel(x, aspect_ratio, local_pe, global_pe, gate):
  b, t, n, d = x.shape
  g2 = jnp.tanh(gate[0].astype(jnp.float32))
  ar = aspect_ratio.astype(jnp.int32)
  h = ar[:, 0]
  w = ar[:, 1]
  wsafe = jnp.maximum(w, 1)
  tt = jnp.arange(T, dtype=jnp.int32)
  rows = tt[None, :] // wsafe[:, None]
  cols = tt[None, :] % wsafe[:, None]
  plane = (rows * T + cols).reshape(BT)
  valid = (tt[None, :] < (h * w)[:, None]).reshape(BT)
  mflat = g2 * valid.astype(jnp.float32)
  p_arr = jnp.tile(plane.reshape(BT, 1), (1, 128))
  m_arr = jnp.tile(mflat.reshape(BT, 1), (1, 128))
  c_arr = jnp.tile((1.0 - g2).reshape(1, 1), (1, 128))
  x3 = x.reshape(BT, N, D)
  g3 = global_pe.reshape(T * T, N, D)
  out_tc = _tc_fused(x, local_pe, g3, plane, m_arr[:, 0],
                     c_arr[0, :1])
  out_sc = _sc_fused(x3, g3, local_pe, p_arr, m_arr, c_arr)
  return _tc_assemble(out_tc, out_sc.reshape(b, t, N - S, d))
